# Initial kernel scaffold; baseline (speedup 1.0000x reference)
import jax
import jax.numpy as jnp
from jax.experimental import pallas as pl


def kernel(time_poc, w0, b0, w1, b1, w2, b2, w3, b3, w4, times_sel, depth, instance_scale_list):
    raise NotImplementedError("write your pallas kernel here")


# speedup vs baseline: 1.0147x; 1.0147x over previous
import jax
import jax.numpy as jnp
from jax.experimental import pallas as pl
from jax.experimental.pallas import tpu as pltpu

_PE = 4
_MAX_TIME = 9.0


def _fused_body(t_ref, inst_ref, poc_ref, w0_ref, b0_ref, w1_ref, b1_ref,
                w2_ref, b2_ref, w3_ref, b3_ref, w4_ref, depth_ref,
                rot_ref, trans_ref, cvd_ref):
    f32 = jnp.float32
    t = t_ref[...]
    inst = inst_ref[...]
    nc = inst.shape[0]

    idx = jnp.clip((t * _MAX_TIME).astype(jnp.int32), 0, nc - 1)
    cam = jax.lax.broadcasted_iota(jnp.int32, (t.shape[0], nc), 1)
    onehot = (idx == cam).astype(f32)
    scale = jnp.dot(onehot, inst, preferred_element_type=f32)
    scale = scale / inst[0:1, 0:1]

    cvd_ref[...] = depth_ref[...] * scale

    @pl.when(pl.program_id(1) == 0)
    def _pose():
        def dot(a, b):
            return jnp.dot(a, b, preferred_element_type=f32)

        ang = t * poc_ref[...]
        emb = jnp.concatenate([t, jnp.sin(ang), jnp.cos(ang)], axis=1)
        h = jnp.maximum(dot(emb, w0_ref[...]) + b0_ref[...], 0.0)
        h = jnp.maximum(dot(h, w1_ref[...]) + b1_ref[...], 0.0)
        g = jnp.concatenate([h, emb], axis=1)
        g = jnp.maximum(dot(g, w2_ref[...]) + b2_ref[...], 0.0)
        g = jnp.maximum(dot(g, w3_ref[...]) + b3_ref[...], 0.0)
        pf = dot(g, w4_ref[...])

        x, y, z = pf[:, 0:1], pf[:, 1:2], pf[:, 2:3]
        cx, sx = jnp.cos(x), jnp.sin(x)
        cy, sy = jnp.cos(y), jnp.sin(y)
        cz, sz = jnp.cos(z), jnp.sin(z)
        rot_ref[...] = jnp.concatenate(
            [cy * cz, -cy * sz, sy,
             sx * sy * cz + cx * sz, -sx * sy * sz + cx * cz, -sx * cy,
             -cx * sy * cz + sx * sz, cx * sy * sz + sx * cz, cx * cy],
            axis=1)
        trans_ref[...] = pf[:, 3:9]


def _ceil_to(n, m):
    return -(-n // m) * m


def kernel(time_poc, w0, b0, w1, b1, w2, b2, w3, b3, w4,
           times_sel, depth, instance_scale_list):
    B = times_sel.shape[0]
    _, H, W = depth.shape
    HW = H * W

    t_in = times_sel.astype(jnp.float32).reshape(B, 1)
    depth2 = depth.astype(jnp.float32).reshape(B, HW)

    tb = min(_ceil_to(B, 8), 128)
    b_pad = _ceil_to(B, tb)
    hw_pad = _ceil_to(HW, 128)
    hw_tile = next(tile for tile in (6144, 4096, 2048, 1024, 512, 256, 128)
                   if hw_pad % tile == 0)

    if b_pad != B:
        t_in = jnp.pad(t_in, ((0, b_pad - B), (0, 0)))
    if b_pad != B or hw_pad != HW:
        depth2 = jnp.pad(depth2, ((0, b_pad - B), (0, hw_pad - HW)))

    whole = lambda a: pl.BlockSpec(a.shape, lambda i, j: (0,) * a.ndim)
    rot9, trans, cvd = pl.pallas_call(
        _fused_body,
        out_shape=(jax.ShapeDtypeStruct((b_pad, 9), jnp.float32),
                   jax.ShapeDtypeStruct((b_pad, 6), jnp.float32),
                   jax.ShapeDtypeStruct((b_pad, hw_pad), jnp.float32)),
        grid=(b_pad // tb, hw_pad // hw_tile),
        in_specs=[pl.BlockSpec((tb, 1), lambda i, j: (i, 0)),
                  whole(instance_scale_list), whole(time_poc),
                  whole(w0), whole(b0), whole(w1), whole(b1),
                  whole(w2), whole(b2), whole(w3), whole(b3), whole(w4),
                  pl.BlockSpec((tb, hw_tile), lambda i, j: (i, j))],
        out_specs=[pl.BlockSpec((tb, 9), lambda i, j: (i, 0)),
                   pl.BlockSpec((tb, 6), lambda i, j: (i, 0)),
                   pl.BlockSpec((tb, hw_tile), lambda i, j: (i, j))],
        compiler_params=pltpu.CompilerParams(
            dimension_semantics=("parallel", "arbitrary")),
    )(t_in, instance_scale_list.astype(jnp.float32),
      time_poc.astype(jnp.float32), w0, b0, w1, b1, w2, b2, w3, b3, w4,
      depth2)

    rot = rot9[:B].reshape(B, 3, 3)
    return rot, trans[:B], cvd[:B, :HW].reshape(B, 1, H, W)


# speedup vs baseline: 2.8178x; 2.7769x over previous
import jax
import jax.numpy as jnp
from jax.experimental import pallas as pl
from jax.experimental.pallas import tpu as pltpu

_PE = 4
_MAX_TIME = 9.0


def _fused_body(t_ref, inst_ref, poc_ref, w0_ref, b0_ref, w1_ref, b1_ref,
                w2_ref, b2_ref, w3_ref, b3_ref, w4_ref, depth_ref,
                rot_ref, trans_ref, cvd_ref):
    f32 = jnp.float32
    t = t_ref[...]
    inst = inst_ref[...]
    nc = inst.shape[0]

    idx = jnp.clip((t * _MAX_TIME).astype(jnp.int32), 0, nc - 1)
    cam = jax.lax.broadcasted_iota(jnp.int32, (t.shape[0], nc), 1)
    onehot = (idx == cam).astype(f32)
    scale = jnp.dot(onehot, inst, preferred_element_type=f32)
    scale = scale / inst[0:1, 0:1]

    d = depth_ref[...]
    out = d * scale.reshape(scale.shape[0], 1, 1)
    cvd_ref[...] = out.reshape(out.shape[0], 1, out.shape[1], out.shape[2])

    @pl.when(pl.program_id(1) == 0)
    def _pose():
        def dot(a, b):
            return jnp.dot(a, b, preferred_element_type=f32)

        ang = t * poc_ref[...]
        emb = jnp.concatenate([t, jnp.sin(ang), jnp.cos(ang)], axis=1)
        h = jnp.maximum(dot(emb, w0_ref[...]) + b0_ref[...], 0.0)
        h = jnp.maximum(dot(h, w1_ref[...]) + b1_ref[...], 0.0)
        g = jnp.concatenate([h, emb], axis=1)
        g = jnp.maximum(dot(g, w2_ref[...]) + b2_ref[...], 0.0)
        g = jnp.maximum(dot(g, w3_ref[...]) + b3_ref[...], 0.0)
        pf = dot(g, w4_ref[...])

        x, y, z = pf[:, 0:1], pf[:, 1:2], pf[:, 2:3]
        cx, sx = jnp.cos(x), jnp.sin(x)
        cy, sy = jnp.cos(y), jnp.sin(y)
        cz, sz = jnp.cos(z), jnp.sin(z)
        rot_ref[...] = jnp.concatenate(
            [cy * cz, -cy * sz, sy,
             sx * sy * cz + cx * sz, -sx * sy * sz + cx * cz, -sx * cy,
             -cx * sy * cz + sx * sz, cx * sy * sz + sx * cz, cx * cy],
            axis=1)
        trans_ref[...] = pf[:, 3:9]


def _ceil_to(n, m):
    return -(-n // m) * m


def kernel(time_poc, w0, b0, w1, b1, w2, b2, w3, b3, w4,
           times_sel, depth, instance_scale_list):
    B = times_sel.shape[0]
    _, H, W = depth.shape

    t_in = times_sel.astype(jnp.float32).reshape(B, 1)
    depth3 = depth.astype(jnp.float32)

    tb = min(_ceil_to(B, 8), 128)
    b_pad = _ceil_to(B, tb)
    h_pad = _ceil_to(H, 8)
    w_pad = _ceil_to(W, 128)
    h_tile = next(t for t in (24, 16, 12, 8) if h_pad % t == 0)

    if b_pad != B:
        t_in = jnp.pad(t_in, ((0, b_pad - B), (0, 0)))
    if b_pad != B or h_pad != H or w_pad != W:
        depth3 = jnp.pad(depth3, ((0, b_pad - B), (0, h_pad - H), (0, w_pad - W)))

    whole = lambda a: pl.BlockSpec(a.shape, lambda i, j: (0,) * a.ndim)
    rot9, trans, cvd = pl.pallas_call(
        _fused_body,
        out_shape=(jax.ShapeDtypeStruct((b_pad, 9), jnp.float32),
                   jax.ShapeDtypeStruct((b_pad, 6), jnp.float32),
                   jax.ShapeDtypeStruct((b_pad, 1, h_pad, w_pad), jnp.float32)),
        grid=(b_pad // tb, h_pad // h_tile),
        in_specs=[pl.BlockSpec((tb, 1), lambda i, j: (i, 0)),
                  whole(instance_scale_list), whole(time_poc),
                  whole(w0), whole(b0), whole(w1), whole(b1),
                  whole(w2), whole(b2), whole(w3), whole(b3), whole(w4),
                  pl.BlockSpec((tb, h_tile, w_pad), lambda i, j: (i, j, 0))],
        out_specs=[pl.BlockSpec((tb, 9), lambda i, j: (i, 0)),
                   pl.BlockSpec((tb, 6), lambda i, j: (i, 0)),
                   pl.BlockSpec((tb, 1, h_tile, w_pad), lambda i, j: (i, 0, j, 0))],
        compiler_params=pltpu.CompilerParams(
            dimension_semantics=("parallel", "arbitrary")),
    )(t_in, instance_scale_list.astype(jnp.float32),
      time_poc.astype(jnp.float32), w0, b0, w1, b1, w2, b2, w3, b3, w4,
      depth3)

    rot = rot9[:B].reshape(B, 3, 3)
    return rot, trans[:B], cvd[:B, :, :H, :W]


# speedup vs baseline: 2.9910x; 1.0615x over previous
import jax
import jax.numpy as jnp
from jax.experimental import pallas as pl
from jax.experimental.pallas import tpu as pltpu

_PE = 4
_MAX_TIME = 9.0


def _fused_body(t_ref, inst_ref, poc_ref, w0_ref, b0_ref, w1_ref, b1_ref,
                w2_ref, b2_ref, w3_ref, b3_ref, w4_ref, depth_ref,
                rot_ref, trans_ref, cvd_ref):
    f32 = jnp.float32
    t = t_ref[...]
    inst = inst_ref[...]
    nc = inst.shape[0]

    idx = jnp.clip((t * _MAX_TIME).astype(jnp.int32), 0, nc - 1)
    cam = jax.lax.broadcasted_iota(jnp.int32, (t.shape[0], nc), 1)
    onehot = (idx == cam).astype(f32)
    scale = jnp.dot(onehot, inst, preferred_element_type=f32)
    scale = scale / inst[0:1, 0:1]

    d = depth_ref[...]
    out = d * scale.reshape(scale.shape[0], 1, 1)
    cvd_ref[...] = out.reshape(out.shape[0], 1, out.shape[1], out.shape[2])

    @pl.when(pl.program_id(1) == 0)
    def _pose():
        def dot(a, b):
            return jnp.dot(a, b, preferred_element_type=f32)

        ang = t * poc_ref[...]
        emb = jnp.concatenate([t, jnp.sin(ang), jnp.cos(ang)], axis=1)
        h = jnp.maximum(dot(emb, w0_ref[...]) + b0_ref[...], 0.0)
        h = jnp.maximum(dot(h, w1_ref[...]) + b1_ref[...], 0.0)
        g = jnp.concatenate([h, emb], axis=1)
        g = jnp.maximum(dot(g, w2_ref[...]) + b2_ref[...], 0.0)
        g = jnp.maximum(dot(g, w3_ref[...]) + b3_ref[...], 0.0)
        pf = dot(g, w4_ref[...])

        x, y, z = pf[:, 0:1], pf[:, 1:2], pf[:, 2:3]
        cx, sx = jnp.cos(x), jnp.sin(x)
        cy, sy = jnp.cos(y), jnp.sin(y)
        cz, sz = jnp.cos(z), jnp.sin(z)
        rot_ref[...] = jnp.concatenate(
            [cy * cz, -cy * sz, sy,
             sx * sy * cz + cx * sz, -sx * sy * sz + cx * cz, -sx * cy,
             -cx * sy * cz + sx * sz, cx * sy * sz + sx * cz, cx * cy],
            axis=1)
        trans_ref[...] = pf[:, 3:9]


def _ceil_to(n, m):
    return -(-n // m) * m


def kernel(time_poc, w0, b0, w1, b1, w2, b2, w3, b3, w4,
           times_sel, depth, instance_scale_list):
    B = times_sel.shape[0]
    _, H, W = depth.shape

    t_in = times_sel.astype(jnp.float32).reshape(B, 1)
    depth3 = depth.astype(jnp.float32)

    tb = min(_ceil_to(B, 8), 128)
    b_pad = _ceil_to(B, tb)
    h_pad = _ceil_to(H, 8)
    w_pad = _ceil_to(W, 128)
    h_tile = next(t for t in (48, 24, 16, 12, 8) if h_pad % t == 0)

    if b_pad != B:
        t_in = jnp.pad(t_in, ((0, b_pad - B), (0, 0)))
    if b_pad != B or h_pad != H or w_pad != W:
        depth3 = jnp.pad(depth3, ((0, b_pad - B), (0, h_pad - H), (0, w_pad - W)))

    whole = lambda a: pl.BlockSpec(a.shape, lambda i, j: (0,) * a.ndim)
    rot9, trans, cvd = pl.pallas_call(
        _fused_body,
        out_shape=(jax.ShapeDtypeStruct((b_pad, 9), jnp.float32),
                   jax.ShapeDtypeStruct((b_pad, 6), jnp.float32),
                   jax.ShapeDtypeStruct((b_pad, 1, h_pad, w_pad), jnp.float32)),
        grid=(b_pad // tb, h_pad // h_tile),
        in_specs=[pl.BlockSpec((tb, 1), lambda i, j: (i, 0)),
                  whole(instance_scale_list), whole(time_poc),
                  whole(w0), whole(b0), whole(w1), whole(b1),
                  whole(w2), whole(b2), whole(w3), whole(b3), whole(w4),
                  pl.BlockSpec((tb, h_tile, w_pad), lambda i, j: (i, j, 0))],
        out_specs=[pl.BlockSpec((tb, 9), lambda i, j: (i, 0)),
                   pl.BlockSpec((tb, 6), lambda i, j: (i, 0)),
                   pl.BlockSpec((tb, 1, h_tile, w_pad), lambda i, j: (i, 0, j, 0))],
        compiler_params=pltpu.CompilerParams(
            dimension_semantics=("parallel", "arbitrary")),
    )(t_in, instance_scale_list.astype(jnp.float32),
      time_poc.astype(jnp.float32), w0, b0, w1, b1, w2, b2, w3, b3, w4,
      depth3)

    rot = rot9[:B].reshape(B, 3, 3)
    return rot, trans[:B], cvd[:B, :, :H, :W]


# speedup vs baseline: 3.2869x; 1.0989x over previous
import jax
import jax.numpy as jnp
from jax.experimental import pallas as pl
from jax.experimental.pallas import tpu as pltpu

_PE = 4
_MAX_TIME = 9.0


def _fused_body(t_ref, inst_ref, poc_ref, w0_ref, b0_ref, w1_ref, b1_ref,
                w2_ref, b2_ref, w3_ref, b3_ref, w4_ref, depth_ref,
                rot_ref, trans_ref, cvd_ref):
    f32 = jnp.float32
    t = t_ref[...]
    inst = inst_ref[...]
    nc = inst.shape[0]

    idx = jnp.clip((t * _MAX_TIME).astype(jnp.int32), 0, nc - 1)
    cam = jax.lax.broadcasted_iota(jnp.int32, (t.shape[0], nc), 1)
    onehot = (idx == cam).astype(f32)
    scale = jnp.dot(onehot, inst, preferred_element_type=f32)
    scale = scale / inst[0:1, 0:1]

    d = depth_ref[...]
    out = d * scale.reshape(scale.shape[0], 1, 1)
    cvd_ref[...] = out.reshape(out.shape[0], 1, out.shape[1], out.shape[2])

    @pl.when(pl.program_id(1) == 0)
    def _pose():
        def dot(a, b):
            return jnp.dot(a, b, preferred_element_type=f32)

        ang = t * poc_ref[...]
        emb = jnp.concatenate([t, jnp.sin(ang), jnp.cos(ang)], axis=1)
        h = jnp.maximum(dot(emb, w0_ref[...]) + b0_ref[...], 0.0)
        h = jnp.maximum(dot(h, w1_ref[...]) + b1_ref[...], 0.0)
        g = jnp.concatenate([h, emb], axis=1)
        g = jnp.maximum(dot(g, w2_ref[...]) + b2_ref[...], 0.0)
        g = jnp.maximum(dot(g, w3_ref[...]) + b3_ref[...], 0.0)
        pf = dot(g, w4_ref[...])

        x, y, z = pf[:, 0:1], pf[:, 1:2], pf[:, 2:3]
        cx, sx = jnp.cos(x), jnp.sin(x)
        cy, sy = jnp.cos(y), jnp.sin(y)
        cz, sz = jnp.cos(z), jnp.sin(z)
        rot_ref[...] = jnp.concatenate(
            [cy * cz, -cy * sz, sy,
             sx * sy * cz + cx * sz, -sx * sy * sz + cx * cz, -sx * cy,
             -cx * sy * cz + sx * sz, cx * sy * sz + sx * cz, cx * cy],
            axis=1)
        trans_ref[...] = pf[:, 3:9]


def _ceil_to(n, m):
    return -(-n // m) * m


def kernel(time_poc, w0, b0, w1, b1, w2, b2, w3, b3, w4,
           times_sel, depth, instance_scale_list):
    B = times_sel.shape[0]
    _, H, W = depth.shape

    t_in = times_sel.astype(jnp.float32).reshape(B, 1)
    depth3 = depth.astype(jnp.float32)

    tb = min(_ceil_to(B, 8), 128)
    b_pad = _ceil_to(B, tb)
    h_pad = _ceil_to(H, 8)
    w_pad = _ceil_to(W, 128)
    h_tile = next(t for t in (96, 48, 24, 16, 12, 8) if h_pad % t == 0)

    if b_pad != B:
        t_in = jnp.pad(t_in, ((0, b_pad - B), (0, 0)))
    if b_pad != B or h_pad != H or w_pad != W:
        depth3 = jnp.pad(depth3, ((0, b_pad - B), (0, h_pad - H), (0, w_pad - W)))

    whole = lambda a: pl.BlockSpec(a.shape, lambda i, j: (0,) * a.ndim)
    rot9, trans, cvd = pl.pallas_call(
        _fused_body,
        out_shape=(jax.ShapeDtypeStruct((b_pad, 9), jnp.float32),
                   jax.ShapeDtypeStruct((b_pad, 6), jnp.float32),
                   jax.ShapeDtypeStruct((b_pad, 1, h_pad, w_pad), jnp.float32)),
        grid=(b_pad // tb, h_pad // h_tile),
        in_specs=[pl.BlockSpec((tb, 1), lambda i, j: (i, 0)),
                  whole(instance_scale_list), whole(time_poc),
                  whole(w0), whole(b0), whole(w1), whole(b1),
                  whole(w2), whole(b2), whole(w3), whole(b3), whole(w4),
                  pl.BlockSpec((tb, h_tile, w_pad), lambda i, j: (i, j, 0))],
        out_specs=[pl.BlockSpec((tb, 9), lambda i, j: (i, 0)),
                   pl.BlockSpec((tb, 6), lambda i, j: (i, 0)),
                   pl.BlockSpec((tb, 1, h_tile, w_pad), lambda i, j: (i, 0, j, 0))],
        compiler_params=pltpu.CompilerParams(
            dimension_semantics=("parallel", "arbitrary")),
    )(t_in, instance_scale_list.astype(jnp.float32),
      time_poc.astype(jnp.float32), w0, b0, w1, b1, w2, b2, w3, b3, w4,
      depth3)

    rot = rot9[:B].reshape(B, 3, 3)
    return rot, trans[:B], cvd[:B, :, :H, :W]


# speedup vs baseline: 3.2983x; 1.0035x over previous
import jax
import jax.numpy as jnp
from jax.experimental import pallas as pl
from jax.experimental.pallas import tpu as pltpu

_PE = 4
_MAX_TIME = 9.0


def _fused_body(t_ref, inst_ref, poc_ref, w0_ref, b0_ref, w1_ref, b1_ref,
                w2_ref, b2_ref, w3_ref, b3_ref, w4_ref, depth_ref,
                rot_ref, trans_ref, cvd_ref):
    f32 = jnp.float32
    t = t_ref[...]
    inst = inst_ref[...]
    nc = inst.shape[0]

    idx = jnp.clip((t * _MAX_TIME).astype(jnp.int32), 0, nc - 1)
    cam = jax.lax.broadcasted_iota(jnp.int32, (t.shape[0], nc), 1)
    onehot = (idx == cam).astype(f32)
    scale = jnp.dot(onehot, inst, preferred_element_type=f32)
    scale = scale / inst[0:1, 0:1]

    d = depth_ref[...]
    out = d * scale.reshape(scale.shape[0], 1, 1)
    cvd_ref[...] = out.reshape(out.shape[0], 1, out.shape[1], out.shape[2])

    @pl.when(pl.program_id(1) == 0)
    def _pose():
        def dot(a, b):
            return jnp.dot(a, b, preferred_element_type=f32)

        ang = t * poc_ref[...]
        emb = jnp.concatenate([t, jnp.sin(ang), jnp.cos(ang)], axis=1)
        h = jnp.maximum(dot(emb, w0_ref[...]) + b0_ref[...], 0.0)
        h = jnp.maximum(dot(h, w1_ref[...]) + b1_ref[...], 0.0)
        g = jnp.concatenate([h, emb], axis=1)
        g = jnp.maximum(dot(g, w2_ref[...]) + b2_ref[...], 0.0)
        g = jnp.maximum(dot(g, w3_ref[...]) + b3_ref[...], 0.0)
        pf = dot(g, w4_ref[...])

        x, y, z = pf[:, 0:1], pf[:, 1:2], pf[:, 2:3]
        cx, sx = jnp.cos(x), jnp.sin(x)
        cy, sy = jnp.cos(y), jnp.sin(y)
        cz, sz = jnp.cos(z), jnp.sin(z)
        rot_ref[...] = jnp.concatenate(
            [cy * cz, -cy * sz, sy,
             sx * sy * cz + cx * sz, -sx * sy * sz + cx * cz, -sx * cy,
             -cx * sy * cz + sx * sz, cx * sy * sz + sx * cz, cx * cy],
            axis=1)
        trans_ref[...] = pf[:, 3:9]


def _ceil_to(n, m):
    return -(-n // m) * m


def kernel(time_poc, w0, b0, w1, b1, w2, b2, w3, b3, w4,
           times_sel, depth, instance_scale_list):
    B = times_sel.shape[0]
    _, H, W = depth.shape

    t_in = times_sel.astype(jnp.float32).reshape(B, 1)
    depth3 = depth.astype(jnp.float32)

    h_pad = _ceil_to(H, 8)
    w_pad = _ceil_to(W, 128)
    budget = 12 << 20
    tb = max(8, min(min(_ceil_to(B, 8), 128),
                    (budget // (h_pad * w_pad * 4)) // 8 * 8))
    b_pad = _ceil_to(B, tb)
    cands = [h_pad] + [t for t in (96, 48, 24, 16, 12, 8) if h_pad % t == 0]
    h_tile = next(t for t in cands
                  if tb * t * w_pad * 4 <= budget or t == cands[-1])

    if b_pad != B:
        t_in = jnp.pad(t_in, ((0, b_pad - B), (0, 0)))
    if b_pad != B or h_pad != H or w_pad != W:
        depth3 = jnp.pad(depth3, ((0, b_pad - B), (0, h_pad - H), (0, w_pad - W)))

    whole = lambda a: pl.BlockSpec(a.shape, lambda i, j: (0,) * a.ndim)
    rot9, trans, cvd = pl.pallas_call(
        _fused_body,
        out_shape=(jax.ShapeDtypeStruct((b_pad, 9), jnp.float32),
                   jax.ShapeDtypeStruct((b_pad, 6), jnp.float32),
                   jax.ShapeDtypeStruct((b_pad, 1, h_pad, w_pad), jnp.float32)),
        grid=(b_pad // tb, h_pad // h_tile),
        in_specs=[pl.BlockSpec((tb, 1), lambda i, j: (i, 0)),
                  whole(instance_scale_list), whole(time_poc),
                  whole(w0), whole(b0), whole(w1), whole(b1),
                  whole(w2), whole(b2), whole(w3), whole(b3), whole(w4),
                  pl.BlockSpec((tb, h_tile, w_pad), lambda i, j: (i, j, 0))],
        out_specs=[pl.BlockSpec((tb, 9), lambda i, j: (i, 0)),
                   pl.BlockSpec((tb, 6), lambda i, j: (i, 0)),
                   pl.BlockSpec((tb, 1, h_tile, w_pad), lambda i, j: (i, 0, j, 0))],
        compiler_params=pltpu.CompilerParams(
            dimension_semantics=("parallel", "arbitrary")),
    )(t_in, instance_scale_list.astype(jnp.float32),
      time_poc.astype(jnp.float32), w0, b0, w1, b1, w2, b2, w3, b3, w4,
      depth3)

    rot = rot9[:B].reshape(B, 3, 3)
    return rot, trans[:B], cvd[:B, :, :H, :W]
